# Initial kernel scaffold; baseline (speedup 1.0000x reference)
import jax
import jax.numpy as jnp
from jax.experimental import pallas as pl


def kernel(new_points_xyzs, new_points_feat, new_pts_feats, ext_pts_info, ext_pts_roi_inds, rois, class_labels, class_pred, W1, b1, W_cls, b_cls, W_reg, b_reg):
    raise NotImplementedError("write your pallas kernel here")


# speedup vs baseline: 1.1635x; 1.1635x over previous
import jax
import jax.numpy as jnp
from jax.experimental import pallas as pl
from jax.experimental.pallas import tpu as pltpu

N_ROIS = 20000
N_PTS = 200000
D_FEAT = 64
D_HID = 64

_C = 400
_NCHUNK = N_PTS // _C
_W = 416
_OUT_PAD = 20416


def _mlp_segmax_body(lo8_ref, xyz_ref, f1_ref, f2_ref, ext_ref, idr_ref, idc_ref,
                     w1a_ref, w1b_ref, w1c_ref, w1d_ref, b1_ref, out_ref):
    i = pl.program_id(0)

    @pl.when(i == 0)
    def _init():
        out_ref[...] = jnp.zeros_like(out_ref)

    h = (jnp.dot(xyz_ref[...], w1a_ref[...], preferred_element_type=jnp.float32)
         + jnp.dot(f1_ref[...], w1b_ref[...], preferred_element_type=jnp.float32)
         + jnp.dot(f2_ref[...], w1c_ref[...], preferred_element_type=jnp.float32)
         + jnp.dot(ext_ref[...], w1d_ref[...], preferred_element_type=jnp.float32)
         + b1_ref[...])
    h = jnp.maximum(h, 0.0)

    idc = idc_ref[0]
    f = h
    s = 1
    while s < _C:
        f_sh = jnp.concatenate([jnp.zeros((s, D_HID), f.dtype), f[:-s, :]], axis=0)
        id_sh = jnp.concatenate(
            [jnp.full((s, 1), -1, idc.dtype), idc[:-s, :]], axis=0)
        f = jnp.where(idc == id_sh, jnp.maximum(f, f_sh), f)
        s *= 2

    idr = idr_ref[0]
    nxt = jnp.concatenate([idr[:, 1:], idr[:, -1:] + 1], axis=1)
    bound = idr != nxt
    lo = lo8_ref[i]
    rel = idr - lo
    rows = jax.lax.broadcasted_iota(jnp.int32, (_W, _C), 0)
    g = jnp.where((rows == rel) & bound, 1.0, 0.0)
    win = jnp.dot(g, f, preferred_element_type=jnp.float32)

    cur = out_ref[pl.ds(lo, _W), :]
    out_ref[pl.ds(lo, _W), :] = jnp.maximum(cur, win)


def _head_body(pooled_ref, rois_ref, wcr_ref, bcr_ref,
               cls_ref, pred_ref, boxes_ref):
    p = pooled_ref[...]
    o8 = jnp.dot(p, wcr_ref[...], preferred_element_type=jnp.float32) + bcr_ref[...]
    cls_ref[...] = jax.nn.sigmoid(o8[:, 0:1])
    pred_ref[...] = o8[:, 1:8]

    rois = rois_ref[...]
    wa = rois[:, 4:5]
    la = rois[:, 5:6]
    ha = rois[:, 6:7]
    ra = rois[:, 7:8]
    xt = o8[:, 1:2]
    yt = o8[:, 2:3]
    zt = o8[:, 3:4]
    wt = o8[:, 4:5]
    lt = o8[:, 5:6]
    ht = o8[:, 6:7]
    rt = o8[:, 7:8]
    diag = jnp.sqrt(la * la + wa * wa)
    wg = jnp.exp(wt) * wa
    lg = jnp.exp(lt) * la
    hg = jnp.exp(ht) * ha
    xg = xt * diag
    yg = yt * diag
    zg = zt * ha + ha * 0.5 - hg * 0.5
    rg = rt + ra
    boxes_ref[...] = jnp.concatenate([xg, yg, zg, wg, lg, hg, rg], axis=1)


def kernel(new_points_xyzs, new_points_feat, new_pts_feats, ext_pts_info,
           ext_pts_roi_inds, rois, class_labels, class_pred,
           W1, b1, W_cls, b_cls, W_reg, b_reg):
    ids = ext_pts_roi_inds.astype(jnp.int32)
    lo8 = (ids[::_C] // 8) * 8
    ids_row3 = ids.reshape(_NCHUNK, 1, _C)
    ids_col3 = ids.reshape(_NCHUNK, _C, 1)

    w1a = W1[0:3]
    w1b = W1[3:3 + D_FEAT]
    w1c = W1[3 + D_FEAT:3 + 2 * D_FEAT]
    w1d = W1[3 + 2 * D_FEAT:]
    b1r = b1.reshape(1, D_HID)

    pooled = pl.pallas_call(
        _mlp_segmax_body,
        grid_spec=pltpu.PrefetchScalarGridSpec(
            num_scalar_prefetch=1,
            grid=(_NCHUNK,),
            in_specs=[
                pl.BlockSpec((_C, 3), lambda i, lo: (i, 0)),
                pl.BlockSpec((_C, D_FEAT), lambda i, lo: (i, 0)),
                pl.BlockSpec((_C, D_FEAT), lambda i, lo: (i, 0)),
                pl.BlockSpec((_C, 12), lambda i, lo: (i, 0)),
                pl.BlockSpec((1, 1, _C), lambda i, lo: (i, 0, 0)),
                pl.BlockSpec((1, _C, 1), lambda i, lo: (i, 0, 0)),
                pl.BlockSpec((3, D_HID), lambda i, lo: (0, 0)),
                pl.BlockSpec((D_FEAT, D_HID), lambda i, lo: (0, 0)),
                pl.BlockSpec((D_FEAT, D_HID), lambda i, lo: (0, 0)),
                pl.BlockSpec((12, D_HID), lambda i, lo: (0, 0)),
                pl.BlockSpec((1, D_HID), lambda i, lo: (0, 0)),
            ],
            out_specs=pl.BlockSpec((_OUT_PAD, D_HID), lambda i, lo: (0, 0)),
        ),
        out_shape=jax.ShapeDtypeStruct((_OUT_PAD, D_HID), jnp.float32),
    )(lo8, new_points_xyzs, new_points_feat, new_pts_feats, ext_pts_info,
      ids_row3, ids_col3, w1a, w1b, w1c, w1d, b1r)

    wcr = jnp.concatenate([W_cls, W_reg], axis=1)
    bcr = jnp.concatenate([b_cls, b_reg]).reshape(1, 8)

    _R = 2000
    cls_score, bbox_pred, boxes3d = pl.pallas_call(
        _head_body,
        grid=(N_ROIS // _R,),
        in_specs=[
            pl.BlockSpec((_R, D_HID), lambda i: (i, 0)),
            pl.BlockSpec((_R, 8), lambda i: (i, 0)),
            pl.BlockSpec((D_HID, 8), lambda i: (0, 0)),
            pl.BlockSpec((1, 8), lambda i: (0, 0)),
        ],
        out_specs=[
            pl.BlockSpec((_R, 1), lambda i: (i, 0)),
            pl.BlockSpec((_R, 7), lambda i: (i, 0)),
            pl.BlockSpec((_R, 7), lambda i: (i, 0)),
        ],
        out_shape=[
            jax.ShapeDtypeStruct((N_ROIS, 1), jnp.float32),
            jax.ShapeDtypeStruct((N_ROIS, 7), jnp.float32),
            jax.ShapeDtypeStruct((N_ROIS, 7), jnp.float32),
        ],
    )(pooled, rois, wcr, bcr)

    roi_boxes = rois[:, 1:]
    return (roi_boxes, bbox_pred, cls_score, boxes3d, class_labels, class_pred)


# speedup vs baseline: 1.7881x; 1.5369x over previous
import functools
import jax
import jax.numpy as jnp
from jax import lax
from jax.experimental import pallas as pl
from jax.experimental.pallas import tpu as pltpu
from jax.experimental.pallas import tpu_sc as plsc

N_ROIS = 20000
N_PTS = 200000
D_FEAT = 64
D_HID = 64

_PC = 2048
_N_PAD = 200704
_NTILE = 32
_BIG = 6256
_SMALL = 6240
_NCH = 10
_CHP = 640
_IDS_ROWS = 12544


def _mlp_body(xyz_ref, f1_ref, f2_ref, ext_ref,
              w1a_ref, w1b_ref, w1c_ref, w1d_ref, b1_ref, h_ref):
    h = (jnp.dot(xyz_ref[...], w1a_ref[...], preferred_element_type=jnp.float32)
         + jnp.dot(f1_ref[...], w1b_ref[...], preferred_element_type=jnp.float32)
         + jnp.dot(f2_ref[...], w1c_ref[...], preferred_element_type=jnp.float32)
         + jnp.dot(ext_ref[...], w1d_ref[...], preferred_element_type=jnp.float32)
         + b1_ref[...])
    h_ref[...] = jnp.maximum(h, 0.0)


def _segmax_sc(h_hbm, ids_hbm, out_hbm, xid_hbm, xval_hbm,
               hbuf, idbuf, fbuf, xv, xi):
    c = lax.axis_index("c")
    s = lax.axis_index("s")
    wid = c * 16 + s
    start = jnp.where(wid < 20, wid * _BIG, 20 * _BIG + (wid - 20) * _SMALL)
    nwin = jnp.where(wid < 20, _BIG // 16, _SMALL // 16)

    zf = jnp.zeros((16,), jnp.float32)
    for k in range(4):
        xv[pl.ds(k * 16, 16)] = zf
    xi[pl.ds(0, 16)] = jnp.zeros((16,), jnp.int32)

    def flush(cur_id, a, fpos, bstart, first_done):
        def head_export(f, b):
            for k in range(4):
                xv[pl.ds(k * 16, 16)] = a[k]
            xi[pl.ds(0, 16)] = jnp.full((16,), cur_id, jnp.int32)
            return f, b

        def append(f, b):
            b2 = jnp.where(f == 0, cur_id, b)
            slot = f & 127
            for k in range(4):
                fbuf[pl.ds(slot * 64 + k * 16, 16)] = a[k]
            return f + 1, b2

        f2, b2 = lax.cond(first_done == 0, head_export, append, fpos, bstart)
        return f2, b2, jnp.int32(1)

    carry = (jnp.int32(-1), jnp.int32(0), jnp.int32(0), jnp.int32(0),
             jnp.int32(0), zf, zf, zf, zf)

    for ci in range(_NCH):
        cstart = pl.multiple_of(start + ci * _CHP, 16)
        base_r = jnp.maximum((cstart >> 4) - 1, 0)
        rowoff = (cstart >> 4) - base_r
        pltpu.sync_copy(h_hbm.at[pl.ds(cstart, _CHP)], hbuf)
        pltpu.sync_copy(ids_hbm.at[pl.ds(pl.multiple_of(base_r * 16, 16), 656)],
                        idbuf)

        if ci == 0:
            iv0 = idbuf[pl.ds(0, 16)]
            prev = jnp.where(wid == 0, jnp.int32(-1), iv0[15])
            carry = (prev,) + carry[1:]

        def wbody(w, carry):
            cur_id, fpos, flushed, bstart, first_done, a0, a1, a2, a3 = carry
            idv = idbuf[pl.ds((rowoff + w) * 16, 16)]
            base = w * 16
            for l in range(16):
                pid = idv[l]
                is_new = pid != cur_id
                fpos, bstart, first_done = lax.cond(
                    is_new,
                    lambda f, b, fd, ci_=cur_id, aa=(a0, a1, a2, a3):
                        flush(ci_, aa, f, b, fd),
                    lambda f, b, fd: (f, b, fd),
                    fpos, bstart, first_done)
                n0 = hbuf[base + l, pl.ds(0, 16)]
                n1 = hbuf[base + l, pl.ds(16, 16)]
                n2 = hbuf[base + l, pl.ds(32, 16)]
                n3 = hbuf[base + l, pl.ds(48, 16)]
                a0 = jnp.where(is_new, n0, jnp.maximum(a0, n0))
                a1 = jnp.where(is_new, n1, jnp.maximum(a1, n1))
                a2 = jnp.where(is_new, n2, jnp.maximum(a2, n2))
                a3 = jnp.where(is_new, n3, jnp.maximum(a3, n3))
                cur_id = pid

            @pl.when(fpos - flushed >= 64)
            def _():
                dst = out_hbm.at[pl.ds(
                    pl.multiple_of((bstart + flushed) * 64, 64), 4096)]
                lax.cond((flushed & 64) == 0,
                         lambda: pltpu.sync_copy(fbuf.at[pl.ds(0, 4096)], dst),
                         lambda: pltpu.sync_copy(fbuf.at[pl.ds(4096, 4096)], dst))

            flushed = jnp.where(fpos - flushed >= 64, flushed + 64, flushed)
            return (cur_id, fpos, flushed, bstart, first_done, a0, a1, a2, a3)

        wlim = jnp.minimum(40, nwin - ci * 40)
        carry = lax.fori_loop(0, wlim, wbody, carry)

    cur_id, fpos, flushed, bstart, first_done, a0, a1, a2, a3 = carry
    fpos, bstart, first_done = flush(
        cur_id, (a0, a1, a2, a3), fpos, bstart, first_done)

    def drain(r, _):
        slot = (flushed + r) & 127
        pltpu.sync_copy(
            fbuf.at[pl.ds(pl.multiple_of(slot * 64, 64), 64)],
            out_hbm.at[pl.ds(
                pl.multiple_of((bstart + flushed + r) * 64, 64), 64)])
        return 0

    lax.fori_loop(0, fpos - flushed, drain, 0)
    pltpu.sync_copy(xv, xval_hbm.at[pl.ds(pl.multiple_of(wid * 64, 64), 64)])
    pltpu.sync_copy(xi, xid_hbm.at[pl.ds(pl.multiple_of(wid * 16, 16), 16)])


def _merge_body(xval_ref, xid_ref, pin_ref, pooled_ref):
    pooled_ref[...] = pin_ref[...]
    rows8 = jax.lax.broadcasted_iota(jnp.int32, (8, D_HID), 0)
    for t in range(_NTILE):
        rid = jnp.maximum(xid_ref[t, 0], 0)
        base = pl.multiple_of((rid // 8) * 8, 8)
        win = pooled_ref[pl.ds(base, 8), :]
        xrow = xval_ref[pl.ds(t, 1), :]
        merged = jnp.where(rows8 == (rid - base), jnp.maximum(win, xrow), win)
        pooled_ref[pl.ds(base, 8), :] = merged


def _head_body(pooled_ref, rois_ref, wcr_ref, bcr_ref,
               cls_ref, pred_ref, boxes_ref):
    p = pooled_ref[...]
    o8 = jnp.dot(p, wcr_ref[...], preferred_element_type=jnp.float32) + bcr_ref[...]
    cls_ref[...] = jax.nn.sigmoid(o8[:, 0:1])
    pred_ref[...] = o8[:, 1:8]
    rois = rois_ref[...]
    wa = rois[:, 4:5]
    la = rois[:, 5:6]
    ha = rois[:, 6:7]
    ra = rois[:, 7:8]
    diag = jnp.sqrt(la * la + wa * wa)
    wg = jnp.exp(o8[:, 4:5]) * wa
    lg = jnp.exp(o8[:, 5:6]) * la
    hg = jnp.exp(o8[:, 6:7]) * ha
    xg = o8[:, 1:2] * diag
    yg = o8[:, 2:3] * diag
    zg = o8[:, 3:4] * ha + ha * 0.5 - hg * 0.5
    rg = o8[:, 7:8] + ra
    boxes_ref[...] = jnp.concatenate([xg, yg, zg, wg, lg, hg, rg], axis=1)


def kernel(new_points_xyzs, new_points_feat, new_pts_feats, ext_pts_info,
           ext_pts_roi_inds, rois, class_labels, class_pred,
           W1, b1, W_cls, b_cls, W_reg, b_reg):
    ids = ext_pts_roi_inds.astype(jnp.int32)
    ids1 = jnp.pad(ids, (0, _IDS_ROWS * 16 - N_PTS),
                   constant_values=N_ROIS - 1)

    w1a = W1[0:3]
    w1b = W1[3:3 + D_FEAT]
    w1c = W1[3 + D_FEAT:3 + 2 * D_FEAT]
    w1d = W1[3 + 2 * D_FEAT:]
    b1r = b1.reshape(1, D_HID)

    h = pl.pallas_call(
        _mlp_body,
        grid=(_N_PAD // _PC,),
        in_specs=[
            pl.BlockSpec((_PC, 3), lambda i: (i, 0)),
            pl.BlockSpec((_PC, D_FEAT), lambda i: (i, 0)),
            pl.BlockSpec((_PC, D_FEAT), lambda i: (i, 0)),
            pl.BlockSpec((_PC, 12), lambda i: (i, 0)),
            pl.BlockSpec((3, D_HID), lambda i: (0, 0)),
            pl.BlockSpec((D_FEAT, D_HID), lambda i: (0, 0)),
            pl.BlockSpec((D_FEAT, D_HID), lambda i: (0, 0)),
            pl.BlockSpec((12, D_HID), lambda i: (0, 0)),
            pl.BlockSpec((1, D_HID), lambda i: (0, 0)),
        ],
        out_specs=pl.BlockSpec((_PC, D_HID), lambda i: (i, 0)),
        out_shape=jax.ShapeDtypeStruct((_N_PAD, D_HID), jnp.float32),
    )(new_points_xyzs, new_points_feat, new_pts_feats, ext_pts_info,
      w1a, w1b, w1c, w1d, b1r)

    mesh = plsc.VectorSubcoreMesh(core_axis_name="c", subcore_axis_name="s")
    segmax = functools.partial(
        pl.kernel, _segmax_sc, mesh=mesh,
        out_type=[
            jax.ShapeDtypeStruct((N_ROIS * D_HID,), jnp.float32),
            jax.ShapeDtypeStruct((_NTILE * 16,), jnp.int32),
            jax.ShapeDtypeStruct((_NTILE * D_HID,), jnp.float32),
        ],
        scratch_types=[
            pltpu.VMEM((_CHP, D_HID), jnp.float32),
            pltpu.VMEM((656,), jnp.int32),
            pltpu.VMEM((128 * D_HID,), jnp.float32),
            pltpu.VMEM((D_HID,), jnp.float32),
            pltpu.VMEM((16,), jnp.int32),
        ],
    )()
    pooled1, xids1, xvals1 = segmax(h, ids1)
    pooled0 = pooled1.reshape(N_ROIS, D_HID)
    xids = xids1.reshape(_NTILE, 16)
    xvals = xvals1.reshape(_NTILE, D_HID)

    pooled = pl.pallas_call(
        _merge_body,
        grid=(1,),
        in_specs=[
            pl.BlockSpec((_NTILE, D_HID), lambda i: (0, 0)),
            pl.BlockSpec(memory_space=pltpu.SMEM),
            pl.BlockSpec((N_ROIS, D_HID), lambda i: (0, 0)),
        ],
        out_specs=pl.BlockSpec((N_ROIS, D_HID), lambda i: (0, 0)),
        out_shape=jax.ShapeDtypeStruct((N_ROIS, D_HID), jnp.float32),
    )(xvals, xids[:, 0:1], pooled0)

    wcr = jnp.concatenate([W_cls, W_reg], axis=1)
    bcr = jnp.concatenate([b_cls, b_reg]).reshape(1, 8)

    _R = 2000
    cls_score, bbox_pred, boxes3d = pl.pallas_call(
        _head_body,
        grid=(N_ROIS // _R,),
        in_specs=[
            pl.BlockSpec((_R, D_HID), lambda i: (i, 0)),
            pl.BlockSpec((_R, 8), lambda i: (i, 0)),
            pl.BlockSpec((D_HID, 8), lambda i: (0, 0)),
            pl.BlockSpec((1, 8), lambda i: (0, 0)),
        ],
        out_specs=[
            pl.BlockSpec((_R, 1), lambda i: (i, 0)),
            pl.BlockSpec((_R, 7), lambda i: (i, 0)),
            pl.BlockSpec((_R, 7), lambda i: (i, 0)),
        ],
        out_shape=[
            jax.ShapeDtypeStruct((N_ROIS, 1), jnp.float32),
            jax.ShapeDtypeStruct((N_ROIS, 7), jnp.float32),
            jax.ShapeDtypeStruct((N_ROIS, 7), jnp.float32),
        ],
    )(pooled, rois, wcr, bcr)

    roi_boxes = rois[:, 1:]
    return (roi_boxes, bbox_pred, cls_score, boxes3d, class_labels, class_pred)


# speedup vs baseline: 1.8820x; 1.0525x over previous
import functools
import jax
import jax.numpy as jnp
from jax import lax
from jax.experimental import pallas as pl
from jax.experimental.pallas import tpu as pltpu
from jax.experimental.pallas import tpu_sc as plsc

N_ROIS = 20000
N_PTS = 200000
D_FEAT = 64
D_HID = 64

_PC = 2048
_N_PAD = 200704
_NTILE = 32
_BIG = 6256
_SMALL = 6240
_NCH = 10
_CHP = 640
_IDS_ROWS = 12544


def _mlp_body(xyz_ref, f1_ref, f2_ref, ext_ref,
              w1a_ref, w1b_ref, w1c_ref, w1d_ref, b1_ref, h_ref):
    h = (jnp.dot(xyz_ref[...], w1a_ref[...], preferred_element_type=jnp.float32)
         + jnp.dot(f1_ref[...], w1b_ref[...], preferred_element_type=jnp.float32)
         + jnp.dot(f2_ref[...], w1c_ref[...], preferred_element_type=jnp.float32)
         + jnp.dot(ext_ref[...], w1d_ref[...], preferred_element_type=jnp.float32)
         + b1_ref[...])
    h_ref[...] = jnp.maximum(h, 0.0)


def _segmax_sc(h_hbm, ids_hbm, out_hbm, xid_hbm, xval_hbm,
               hbuf, idbuf, fbuf, xv, xi):
    c = lax.axis_index("c")
    s = lax.axis_index("s")
    wid = c * 16 + s
    start = jnp.where(wid < 20, wid * _BIG, 20 * _BIG + (wid - 20) * _SMALL)
    nwin = jnp.where(wid < 20, _BIG // 16, _SMALL // 16)

    zf = jnp.zeros((16,), jnp.float32)
    for k in range(4):
        xv[pl.ds(k * 16, 16)] = zf
    xi[pl.ds(0, 16)] = jnp.zeros((16,), jnp.int32)

    carry = None

    for ci in range(_NCH):
        cstart = pl.multiple_of(start + ci * _CHP, 16)
        base_r = jnp.maximum((cstart >> 4) - 1, 0)
        rowoff = (cstart >> 4) - base_r
        pltpu.sync_copy(h_hbm.at[pl.ds(cstart, _CHP)], hbuf)
        pltpu.sync_copy(ids_hbm.at[pl.ds(pl.multiple_of(base_r * 16, 16), 656)],
                        idbuf)

        if ci == 0:
            iv0 = idbuf[pl.ds(0, 16)]
            ivf = idbuf[pl.ds(rowoff * 16, 16)]
            id0 = ivf[0]
            prev = jnp.where(wid == 0, jnp.int32(-1), iv0[15])
            clean = (prev != id0).astype(jnp.int32)
            bstart = jnp.where(clean == 1, id0, id0 + 1)
            xi[pl.ds(0, 16)] = jnp.full((16,), id0, jnp.int32)
            carry = (prev, jnp.int32(-1), jnp.int32(0), clean, bstart,
                     zf, zf, zf, zf)

        def wbody(w, carry):
            cur_id, fpos, flushed, exported, bstart, a0, a1, a2, a3 = carry
            idv = idbuf[pl.ds((rowoff + w) * 16, 16)]
            base = w * 16
            for l in range(16):
                pid = idv[l]
                is_new = pid != cur_id
                fpos = fpos + is_new.astype(jnp.int32)
                slot = fpos & 127
                n0 = hbuf[base + l, pl.ds(0, 16)]
                n1 = hbuf[base + l, pl.ds(16, 16)]
                n2 = hbuf[base + l, pl.ds(32, 16)]
                n3 = hbuf[base + l, pl.ds(48, 16)]
                a0 = jnp.where(is_new, n0, jnp.maximum(a0, n0))
                a1 = jnp.where(is_new, n1, jnp.maximum(a1, n1))
                a2 = jnp.where(is_new, n2, jnp.maximum(a2, n2))
                a3 = jnp.where(is_new, n3, jnp.maximum(a3, n3))
                sbase = slot * 64
                fbuf[pl.ds(sbase, 16)] = a0
                fbuf[pl.ds(sbase + 16, 16)] = a1
                fbuf[pl.ds(sbase + 32, 16)] = a2
                fbuf[pl.ds(sbase + 48, 16)] = a3
                cur_id = pid

            @pl.when((exported == 0) & (fpos >= 0))
            def _():
                for k in range(4):
                    xv[pl.ds(k * 16, 16)] = fbuf[pl.ds(127 * 64 + k * 16, 16)]

            exported = jnp.where((exported == 0) & (fpos >= 0), 1, exported)

            @pl.when(fpos - flushed >= 64)
            def _():
                dst = out_hbm.at[pl.ds(
                    pl.multiple_of((bstart + flushed) * 64, 64), 4096)]
                lax.cond((flushed & 64) == 0,
                         lambda: pltpu.sync_copy(fbuf.at[pl.ds(0, 4096)], dst),
                         lambda: pltpu.sync_copy(fbuf.at[pl.ds(4096, 4096)], dst))

            flushed = jnp.where(fpos - flushed >= 64, flushed + 64, flushed)
            return (cur_id, fpos, flushed, exported, bstart, a0, a1, a2, a3)

        wlim = jnp.minimum(40, nwin - ci * 40)
        carry = lax.fori_loop(0, wlim, wbody, carry)

    cur_id, fpos, flushed, exported, bstart, a0, a1, a2, a3 = carry

    @pl.when(exported == 0)
    def _():
        for k in range(4):
            xv[pl.ds(k * 16, 16)] = fbuf[pl.ds(127 * 64 + k * 16, 16)]

    def drain(r, _):
        slot = (flushed + r) & 127
        pltpu.sync_copy(
            fbuf.at[pl.ds(pl.multiple_of(slot * 64, 64), 64)],
            out_hbm.at[pl.ds(
                pl.multiple_of((bstart + flushed + r) * 64, 64), 64)])
        return 0

    ndrain = jnp.where(fpos >= flushed, fpos - flushed + 1, 0)
    lax.fori_loop(0, ndrain, drain, 0)
    pltpu.sync_copy(xv, xval_hbm.at[pl.ds(pl.multiple_of(wid * 64, 64), 64)])
    pltpu.sync_copy(xi, xid_hbm.at[pl.ds(pl.multiple_of(wid * 16, 16), 16)])


def _head_body(pooled_ref, rois_ref, wcr_ref, bcr_ref, xval_ref, xid_ref,
               cls_ref, pred_ref, boxes_ref, ps_ref):
    i = pl.program_id(0)
    nrows = ps_ref.shape[0]
    ps_ref[...] = pooled_ref[...]
    rows8 = jax.lax.broadcasted_iota(jnp.int32, (8, D_HID), 0)
    for t in range(_NTILE):
        rid = xid_ref[t, 0]
        inb = (rid >= i * nrows) & (rid < (i + 1) * nrows)

        @pl.when(inb)
        def _(rid=rid, t=t):
            rel = rid - i * nrows
            base = pl.multiple_of((rel // 8) * 8, 8)
            win = ps_ref[pl.ds(base, 8), :]
            xrow = xval_ref[pl.ds(t, 1), :]
            ps_ref[pl.ds(base, 8), :] = jnp.where(
                rows8 == (rel - base), jnp.maximum(win, xrow), win)

    p = ps_ref[...]
    o8 = jnp.dot(p, wcr_ref[...], preferred_element_type=jnp.float32) + bcr_ref[...]
    cls_ref[...] = jax.nn.sigmoid(o8[:, 0:1])
    pred_ref[...] = o8[:, 1:8]
    rois = rois_ref[...]
    wa = rois[:, 4:5]
    la = rois[:, 5:6]
    ha = rois[:, 6:7]
    ra = rois[:, 7:8]
    diag = jnp.sqrt(la * la + wa * wa)
    wg = jnp.exp(o8[:, 4:5]) * wa
    lg = jnp.exp(o8[:, 5:6]) * la
    hg = jnp.exp(o8[:, 6:7]) * ha
    xg = o8[:, 1:2] * diag
    yg = o8[:, 2:3] * diag
    zg = o8[:, 3:4] * ha + ha * 0.5 - hg * 0.5
    rg = o8[:, 7:8] + ra
    boxes_ref[...] = jnp.concatenate([xg, yg, zg, wg, lg, hg, rg], axis=1)


def kernel(new_points_xyzs, new_points_feat, new_pts_feats, ext_pts_info,
           ext_pts_roi_inds, rois, class_labels, class_pred,
           W1, b1, W_cls, b_cls, W_reg, b_reg):
    ids = ext_pts_roi_inds.astype(jnp.int32)
    ids1 = jnp.pad(ids, (0, _IDS_ROWS * 16 - N_PTS),
                   constant_values=N_ROIS - 1)

    w1a = W1[0:3]
    w1b = W1[3:3 + D_FEAT]
    w1c = W1[3 + D_FEAT:3 + 2 * D_FEAT]
    w1d = W1[3 + 2 * D_FEAT:]
    b1r = b1.reshape(1, D_HID)

    h = pl.pallas_call(
        _mlp_body,
        grid=(_N_PAD // _PC,),
        in_specs=[
            pl.BlockSpec((_PC, 3), lambda i: (i, 0)),
            pl.BlockSpec((_PC, D_FEAT), lambda i: (i, 0)),
            pl.BlockSpec((_PC, D_FEAT), lambda i: (i, 0)),
            pl.BlockSpec((_PC, 12), lambda i: (i, 0)),
            pl.BlockSpec((3, D_HID), lambda i: (0, 0)),
            pl.BlockSpec((D_FEAT, D_HID), lambda i: (0, 0)),
            pl.BlockSpec((D_FEAT, D_HID), lambda i: (0, 0)),
            pl.BlockSpec((12, D_HID), lambda i: (0, 0)),
            pl.BlockSpec((1, D_HID), lambda i: (0, 0)),
        ],
        out_specs=pl.BlockSpec((_PC, D_HID), lambda i: (i, 0)),
        out_shape=jax.ShapeDtypeStruct((_N_PAD, D_HID), jnp.float32),
    )(new_points_xyzs, new_points_feat, new_pts_feats, ext_pts_info,
      w1a, w1b, w1c, w1d, b1r)

    mesh = plsc.VectorSubcoreMesh(core_axis_name="c", subcore_axis_name="s")
    segmax = functools.partial(
        pl.kernel, _segmax_sc, mesh=mesh,
        out_type=[
            jax.ShapeDtypeStruct((N_ROIS * D_HID,), jnp.float32),
            jax.ShapeDtypeStruct((_NTILE * 16,), jnp.int32),
            jax.ShapeDtypeStruct((_NTILE * D_HID,), jnp.float32),
        ],
        scratch_types=[
            pltpu.VMEM((_CHP, D_HID), jnp.float32),
            pltpu.VMEM((656,), jnp.int32),
            pltpu.VMEM((128 * D_HID,), jnp.float32),
            pltpu.VMEM((D_HID,), jnp.float32),
            pltpu.VMEM((16,), jnp.int32),
        ],
    )()
    pooled1, xids1, xvals1 = segmax(h, ids1)
    pooled = pooled1.reshape(N_ROIS, D_HID)
    xids = xids1.reshape(_NTILE, 16)
    xvals = xvals1.reshape(_NTILE, D_HID)

    wcr = jnp.concatenate([W_cls, W_reg], axis=1)
    bcr = jnp.concatenate([b_cls, b_reg]).reshape(1, 8)

    _R = 2000
    cls_score, bbox_pred, boxes3d = pl.pallas_call(
        _head_body,
        grid=(N_ROIS // _R,),
        in_specs=[
            pl.BlockSpec((_R, D_HID), lambda i: (i, 0)),
            pl.BlockSpec((_R, 8), lambda i: (i, 0)),
            pl.BlockSpec((D_HID, 8), lambda i: (0, 0)),
            pl.BlockSpec((1, 8), lambda i: (0, 0)),
            pl.BlockSpec((_NTILE, D_HID), lambda i: (0, 0)),
            pl.BlockSpec(memory_space=pltpu.SMEM),
        ],
        out_specs=[
            pl.BlockSpec((_R, 1), lambda i: (i, 0)),
            pl.BlockSpec((_R, 7), lambda i: (i, 0)),
            pl.BlockSpec((_R, 7), lambda i: (i, 0)),
        ],
        out_shape=[
            jax.ShapeDtypeStruct((N_ROIS, 1), jnp.float32),
            jax.ShapeDtypeStruct((N_ROIS, 7), jnp.float32),
            jax.ShapeDtypeStruct((N_ROIS, 7), jnp.float32),
        ],
        scratch_shapes=[pltpu.VMEM((_R, D_HID), jnp.float32)],
    )(pooled, rois, wcr, bcr, xvals, xids[:, 0:1])

    roi_boxes = rois[:, 1:]
    return (roi_boxes, bbox_pred, cls_score, boxes3d, class_labels, class_pred)
